# baseline (speedup 1.0000x reference)
import functools

import jax
import jax.numpy as jnp
from jax import lax
from jax.experimental import pallas as pl
from jax.experimental.pallas import tpu as pltpu
from jax.experimental.pallas import tpu_sc as plsc

NUM_CORES = 2
NUM_SUBCORES = 16
NUM_WORKERS = NUM_CORES * NUM_SUBCORES

SC_ROWS = 2048
TC_BM = 512


def _sc_gather(y, embeddings, rows, dim):
    b_per_w = rows // NUM_WORKERS
    chunk = min(16, b_per_w)
    n_chunks = b_per_w // chunk
    n_bufs = min(4, n_chunks)

    mesh = plsc.VectorSubcoreMesh(core_axis_name="c", subcore_axis_name="s", num_cores=NUM_CORES)

    @functools.partial(
        pl.kernel,
        mesh=mesh,
        out_type=jax.ShapeDtypeStruct((rows, dim), jnp.float32),
        scratch_types=[
            pltpu.VMEM((b_per_w,), jnp.int32),
        ]
        + [pltpu.VMEM((chunk, dim), jnp.float32) for _ in range(n_bufs)]
        + [
            pltpu.SemaphoreType.DMA,
            pltpu.SemaphoreType.DMA,
        ],
    )
    def k(table_hbm, idx_hbm, out_hbm, idx_v, *rest):
        bufs = rest[:n_bufs]
        gsem, ssem = rest[n_bufs:]
        wid = lax.axis_index("s") * NUM_CORES + lax.axis_index("c")
        base = wid * b_per_w
        pltpu.sync_copy(idx_hbm.at[pl.ds(base, b_per_w)], idx_v)

        def gather(c, buf):
            return pltpu.make_async_copy(
                table_hbm.at[idx_v.at[pl.ds(c * chunk, chunk)]], buf, gsem
            )

        def store(c, buf):
            return pltpu.make_async_copy(
                buf, out_hbm.at[pl.ds(base + c * chunk, chunk)], ssem
            )

        for c in range(min(n_bufs, n_chunks)):
            gather(c, bufs[c % n_bufs]).start()
        for c in range(n_chunks):
            buf = bufs[c % n_bufs]
            gather(c, buf).wait()
            store(c, buf).start()
            if c + n_bufs < n_chunks:
                store(c, buf).wait()
                gather(c + n_bufs, buf).start()
        for c in range(max(0, n_chunks - n_bufs), n_chunks):
            store(c, bufs[c % n_bufs]).wait()

    return k(embeddings, y)


def _tc_body(y_ref, tab_ref, o_ref, tab_bf16):
    @pl.when(pl.program_id(0) == 0)
    def _():
        tab_bf16[...] = tab_ref[...].astype(jnp.bfloat16)

    yb = y_ref[...]
    ks = lax.broadcasted_iota(jnp.int32, (TC_BM, tab_ref.shape[0]), 1)
    onehot = (yb == ks).astype(jnp.bfloat16)
    o_ref[...] = jnp.dot(onehot, tab_bf16[...],
                         preferred_element_type=jnp.float32)


def _tc_gather(y, table, batch, rows, dim):
    vocab = table.shape[0]
    skip = (batch - rows) // TC_BM
    return pl.pallas_call(
        _tc_body,
        grid=(rows // TC_BM,),
        in_specs=[
            pl.BlockSpec((TC_BM, 1), lambda i: (i + skip, 0)),
            pl.BlockSpec((vocab, dim), lambda i: (0, 0)),
        ],
        out_specs=pl.BlockSpec((TC_BM, dim), lambda i: (i + skip, 0)),
        out_shape=jax.ShapeDtypeStruct((batch, dim), jnp.float32),
        scratch_shapes=[pltpu.VMEM((vocab, dim), jnp.bfloat16)],
        compiler_params=pltpu.CompilerParams(
            dimension_semantics=("arbitrary",)),
    )(y.reshape(-1, 1), table)


@functools.partial(jax.jit, static_argnames=("batch", "dim", "vocab"))
def _embed_lookup(y, embeddings, batch, dim, vocab):
    sc_out = _sc_gather(y, embeddings, SC_ROWS, dim)
    out = _tc_gather(y, embeddings, batch, batch - SC_ROWS, dim)
    return lax.dynamic_update_slice(out, sc_out, (0, 0))


def kernel(y, embeddings):
    batch = y.shape[0]
    vocab, dim = embeddings.shape
    return _embed_lookup(y.astype(jnp.int32), embeddings, batch, dim, vocab)


# speedup vs baseline: 1.1665x; 1.1665x over previous
import functools

import jax
import jax.numpy as jnp
from jax import lax
from jax.experimental import pallas as pl
from jax.experimental.pallas import tpu as pltpu
from jax.experimental.pallas import tpu_sc as plsc

NUM_CORES = 2
NUM_SUBCORES = 16
NUM_WORKERS = NUM_CORES * NUM_SUBCORES

SC_ROWS = 1024
TC_BM = 512


def _sc_gather(y, embeddings, rows, dim):
    b_per_w = rows // NUM_WORKERS
    chunk = min(16, b_per_w)
    n_chunks = b_per_w // chunk
    n_bufs = min(4, n_chunks)

    mesh = plsc.VectorSubcoreMesh(core_axis_name="c", subcore_axis_name="s")

    @functools.partial(
        pl.kernel,
        mesh=mesh,
        out_type=jax.ShapeDtypeStruct((rows, dim), jnp.float32),
        scratch_types=[
            pltpu.VMEM((b_per_w,), jnp.int32),
        ]
        + [pltpu.VMEM((chunk, dim), jnp.float32) for _ in range(n_bufs)]
        + [
            pltpu.SemaphoreType.DMA,
            pltpu.SemaphoreType.DMA,
        ],
    )
    def k(table_hbm, idx_hbm, out_hbm, idx_v, *rest):
        bufs = rest[:n_bufs]
        gsem, ssem = rest[n_bufs:]
        wid = lax.axis_index("s") * NUM_CORES + lax.axis_index("c")
        base = wid * b_per_w
        pltpu.sync_copy(idx_hbm.at[pl.ds(base, b_per_w)], idx_v)

        def gather(c, buf):
            return pltpu.make_async_copy(
                table_hbm.at[idx_v.at[pl.ds(c * chunk, chunk)]], buf, gsem
            )

        def store(c, buf):
            return pltpu.make_async_copy(
                buf, out_hbm.at[pl.ds(base + c * chunk, chunk)], ssem
            )

        for c in range(min(n_bufs, n_chunks)):
            gather(c, bufs[c % n_bufs]).start()
        for c in range(n_chunks):
            buf = bufs[c % n_bufs]
            gather(c, buf).wait()
            store(c, buf).start()
            if c + n_bufs < n_chunks:
                store(c, buf).wait()
                gather(c + n_bufs, buf).start()
        for c in range(max(0, n_chunks - n_bufs), n_chunks):
            store(c, bufs[c % n_bufs]).wait()

    return k(embeddings, y)


def _tc_body(y_ref, tab_ref, o_ref, tab_bf16):
    @pl.when(pl.program_id(0) == 0)
    def _():
        tab_bf16[...] = tab_ref[...].astype(jnp.bfloat16)

    vocab = tab_ref.shape[0]
    yb = y_ref[0, 0, :]
    ks = lax.broadcasted_iota(jnp.int32, (vocab, TC_BM), 0)
    onehot_t = (yb[None, :] == ks).astype(jnp.bfloat16)
    o_ref[...] = lax.dot_general(
        onehot_t, tab_bf16[...],
        dimension_numbers=(((0,), (0,)), ((), ())),
        preferred_element_type=jnp.float32,
    )


def _tc_gather(y, table, batch, rows, dim):
    vocab = table.shape[0]
    skip = (batch - rows) // TC_BM
    return pl.pallas_call(
        _tc_body,
        grid=(rows // TC_BM,),
        in_specs=[
            pl.BlockSpec((1, 1, TC_BM), lambda i: (i + skip, 0, 0)),
            pl.BlockSpec((vocab, dim), lambda i: (0, 0)),
        ],
        out_specs=pl.BlockSpec((TC_BM, dim), lambda i: (i + skip, 0)),
        out_shape=jax.ShapeDtypeStruct((batch, dim), jnp.float32),
        scratch_shapes=[pltpu.VMEM((vocab, dim), jnp.bfloat16)],
        compiler_params=pltpu.CompilerParams(
            dimension_semantics=("arbitrary",)),
    )(y.reshape(-1, 1, TC_BM), table)


def _merge_body(sc_ref, tc_ref, o_ref):
    o_ref[...] = sc_ref[...]


def _merge(sc_out, tc_full, rows, dim):
    return pl.pallas_call(
        _merge_body,
        grid=(1,),
        in_specs=[
            pl.BlockSpec((rows, dim), lambda i: (0, 0)),
            pl.BlockSpec(memory_space=pl.ANY),
        ],
        out_specs=pl.BlockSpec((rows, dim), lambda i: (0, 0)),
        out_shape=jax.ShapeDtypeStruct(tc_full.shape, jnp.float32),
        input_output_aliases={1: 0},
    )(sc_out, tc_full)


@functools.partial(jax.jit, static_argnames=("batch", "dim", "vocab"))
def _embed_lookup(y, embeddings, batch, dim, vocab):
    sc_out = _sc_gather(y, embeddings, SC_ROWS, dim)
    out = _tc_gather(y, embeddings, batch, batch - SC_ROWS, dim)
    return _merge(sc_out, out, SC_ROWS, dim)


def kernel(y, embeddings):
    batch = y.shape[0]
    vocab, dim = embeddings.shape
    return _embed_lookup(y.astype(jnp.int32), embeddings, batch, dim, vocab)
